# Initial kernel scaffold; baseline (speedup 1.0000x reference)
import jax
import jax.numpy as jnp
from jax.experimental import pallas as pl


def kernel(x, edge_index, W1, b1, W2, b2):
    raise NotImplementedError("write your pallas kernel here")


# speedup vs baseline: 17.0038x; 17.0038x over previous
import functools

import jax
import jax.numpy as jnp
from jax import lax
from jax.experimental import pallas as pl
from jax.experimental.pallas import tpu as pltpu
from jax.experimental.pallas import tpu_sc as plsc

N = 10000
E = 640000
D_IN = 128
D_H1 = 128
D_H2 = 64

NC = 2
NS = 16
NW = NC * NS

CH = 128
NT = -(-E // (NW * CH))
EP = NW * NT * CH
NP = 10240
RP = NP // NS

_MESH = plsc.VectorSubcoreMesh(core_axis_name="c", subcore_axis_name="s")


def _deg_body(dst, zvec, out, idx_v, ones_v, deg_sh):
    c = lax.axis_index("c")
    s = lax.axis_index("s")
    w = s * NC + c
    for i in range(CH // 16):
        ones_v[pl.ds(i * 16, 16)] = jnp.ones((16,), jnp.float32)
    pltpu.sync_copy(zvec.at[pl.ds(s * RP, RP)], deg_sh.at[pl.ds(s * RP, RP)])
    plsc.subcore_barrier()
    base = w * NT * CH

    def step(t, carry):
        e = base + t * CH
        pltpu.sync_copy(dst.at[pl.ds(e, CH)], idx_v.at[0])
        pltpu.sync_copy(ones_v, deg_sh.at[idx_v.at[0]], add=True)
        return carry

    lax.fori_loop(0, NT, step, 0)
    plsc.subcore_barrier()
    pltpu.sync_copy(deg_sh.at[pl.ds(s * RP, RP)],
                    out.at[pl.ds(c * NP + s * RP, RP)])


_deg_kernel = functools.partial(
    pl.kernel,
    out_type=jax.ShapeDtypeStruct((NC * NP,), jnp.float32),
    mesh=_MESH,
    scratch_types=[
        pltpu.VMEM((1, CH), jnp.int32),
        pltpu.VMEM((CH,), jnp.float32),
        pltpu.VMEM_SHARED((NP,), jnp.float32),
    ],
)(_deg_body)


def _scatter_body(src_ix, dst_ix, zrows, g, out, idx_v, rows_v, acc_sh, sem):
    c = lax.axis_index("c")
    s = lax.axis_index("s")
    w = s * NC + c
    pltpu.sync_copy(zrows.at[pl.ds(s * RP, RP)], acc_sh.at[pl.ds(s * RP, RP)])
    plsc.subcore_barrier()
    base = w * NT * CH

    def step(t, carry):
        e = base + t * CH
        pltpu.sync_copy(src_ix.at[pl.ds(e, CH)], idx_v.at[0])
        pltpu.sync_copy(dst_ix.at[pl.ds(e, CH)], idx_v.at[1])
        pltpu.async_copy(g.at[idx_v.at[0]], rows_v, sem).wait()
        pltpu.sync_copy(rows_v, acc_sh.at[idx_v.at[1]], add=True)
        return carry

    lax.fori_loop(0, NT, step, 0)
    plsc.subcore_barrier()
    pltpu.sync_copy(acc_sh.at[pl.ds(s * RP, RP)], out.at[c, pl.ds(s * RP, RP)])


def _make_scatter_kernel(d):
    return functools.partial(
        pl.kernel,
        out_type=jax.ShapeDtypeStruct((NC, NP, d), jnp.float32),
        mesh=_MESH,
        scratch_types=[
            pltpu.VMEM((2, CH), jnp.int32),
            pltpu.VMEM((CH, d), jnp.float32),
            pltpu.VMEM_SHARED((NP, d), jnp.float32),
            pltpu.SemaphoreType.DMA,
        ],
    )(_scatter_body)


_scatter_128 = _make_scatter_kernel(D_H1)

RB = 1024


def _mm_scale_body(x_ref, w_ref, dinv_ref, o_ref):
    o_ref[...] = (
        jnp.dot(x_ref[...], w_ref[...], preferred_element_type=jnp.float32)
        * dinv_ref[...]
    )


def _mid_body(p0_ref, p1_ref, g_ref, dinv_ref, b_ref, o_ref):
    h = jnp.maximum(
        (p0_ref[...] + p1_ref[...] + g_ref[...]) * dinv_ref[...] + b_ref[...],
        0.0,
    )
    o_ref[...] = h * dinv_ref[...]


def _final_body(p0_ref, p1_ref, g_ref, dinv_ref, b_ref, w_ref, o_ref):
    acc = p0_ref[...] + p1_ref[...] + g_ref[...]
    o_ref[...] = jnp.maximum(
        jnp.dot(acc, w_ref[...], preferred_element_type=jnp.float32)
        * dinv_ref[...]
        + b_ref[...],
        0.0,
    )


def _rows(bs):
    return pl.BlockSpec((RB, bs), lambda i: (i, 0))


def _full(r, c):
    return pl.BlockSpec((r, c), lambda i: (0, 0))


def _mm_scale(x, w, dinv):
    din, dout = w.shape
    return pl.pallas_call(
        _mm_scale_body,
        grid=(NP // RB,),
        in_specs=[_rows(din), _full(din, dout), _rows(1)],
        out_specs=_rows(dout),
        out_shape=jax.ShapeDtypeStruct((NP, dout), jnp.float32),
    )(x, w, dinv)


def _mid(p0, p1, g, dinv, b):
    d = g.shape[1]
    return pl.pallas_call(
        _mid_body,
        grid=(NP // RB,),
        in_specs=[_rows(d), _rows(d), _rows(d), _rows(1), _full(1, d)],
        out_specs=_rows(d),
        out_shape=jax.ShapeDtypeStruct((NP, d), jnp.float32),
    )(p0, p1, g, dinv, b)


def _final(p0, p1, g, dinv, b, w):
    din, dout = w.shape
    return pl.pallas_call(
        _final_body,
        grid=(NP // RB,),
        in_specs=[_rows(din), _rows(din), _rows(din), _rows(1),
                  _full(1, dout), _full(din, dout)],
        out_specs=_rows(dout),
        out_shape=jax.ShapeDtypeStruct((NP, dout), jnp.float32),
    )(p0, p1, g, dinv, b, w)


def kernel(x, edge_index, W1, b1, W2, b2):
    x_p = jnp.zeros((NP, D_IN), jnp.float32).at[:N].set(x)
    pad = (jnp.arange(EP - E, dtype=jnp.int32) % (NP - N)) + N
    src_p = jnp.concatenate([edge_index[0], pad])
    dst_p = jnp.concatenate([edge_index[1], pad])

    zvec = jnp.zeros((NP,), jnp.float32)
    z128 = jnp.zeros((NP, D_H1), jnp.float32)

    degp = _deg_kernel(dst_p, zvec).reshape(NC, NP)
    dinv = lax.rsqrt(degp[0] + degp[1] + 1.0).reshape(NP, 1)

    g1 = _mm_scale(x_p, W1, dinv)
    p1 = _scatter_128(src_p, dst_p, z128, g1)
    u = _mid(p1[0], p1[1], g1, dinv, b1.reshape(1, D_H1))
    p2 = _scatter_128(src_p, dst_p, z128, u)
    z = _final(p2[0], p2[1], u, dinv, b2.reshape(1, D_H2), W2)
    return z[:N]


# speedup vs baseline: 25.4299x; 1.4955x over previous
import functools

import jax
import jax.numpy as jnp
from jax import lax
from jax.experimental import pallas as pl
from jax.experimental.pallas import tpu as pltpu
from jax.experimental.pallas import tpu_sc as plsc

N = 10000
E = 640000
D_IN = 128
D_H1 = 128
D_H2 = 64

NC = 2
NS = 16
NW = NC * NS

CH = 128
G = 2
NT = 160
EP = NW * NT * CH
EPA = EP
NP = 10240
RP = NP // NS

_MESH = plsc.VectorSubcoreMesh(core_axis_name="c", subcore_axis_name="s")


def _deg_body(dst, zvec, out, idx_v, ones_v, deg_sh, *sems):
    sem_i = sems[:4]
    sem_s = sems[4:]
    c = lax.axis_index("c")
    s = lax.axis_index("s")
    w = s * NC + c
    for i in range(CH // 16):
        ones_v[pl.ds(i * 16, 16)] = jnp.ones((16,), jnp.float32)
    pltpu.sync_copy(zvec.at[pl.ds(s * RP, RP)], deg_sh.at[pl.ds(s * RP, RP)])
    plsc.subcore_barrier()
    base = w * NT * CH

    GD = 4

    def group(tt, carry):
        idx_d = []
        for k in range(GD):
            e = base + (tt * GD + k) * CH
            idx_d.append(pltpu.async_copy(
                dst.at[pl.ds(e, CH)], idx_v.at[k], sem_i[k]))
        s_d = []
        for k in range(GD):
            idx_d[k].wait()
            s_d.append(pltpu.async_copy(
                ones_v, deg_sh.at[idx_v.at[k]], sem_s[k], add=True))
        for k in range(GD):
            s_d[k].wait()
        return carry

    lax.fori_loop(0, NT // GD, group, 0)
    plsc.subcore_barrier()
    pltpu.sync_copy(deg_sh.at[pl.ds(s * RP, RP)],
                    out.at[pl.ds(c * NP + s * RP, RP)])


_deg_kernel = functools.partial(
    pl.kernel,
    out_type=jax.ShapeDtypeStruct((NC * NP,), jnp.float32),
    mesh=_MESH,
    scratch_types=[
        pltpu.VMEM((4, CH), jnp.int32),
        pltpu.VMEM((CH,), jnp.float32),
        pltpu.VMEM_SHARED((NP,), jnp.float32),
    ] + [pltpu.SemaphoreType.DMA] * 8,
)(_deg_body)


def _scatter_body(src_ix, dst_ix, zrows, g, out, idx_v, rows_v, acc_sh, *sems):
    sem_i = sems[:G]
    sem_g = sems[G:2 * G]
    sem_s = sems[2 * G:3 * G]
    c = lax.axis_index("c")
    s = lax.axis_index("s")
    w = s * NC + c
    pltpu.sync_copy(zrows.at[pl.ds(s * RP, RP)], acc_sh.at[pl.ds(s * RP, RP)])
    plsc.subcore_barrier()
    base = w * NT * CH

    def group(tt, carry):
        idx_d = []
        for k in range(G):
            e = base + (tt * G + k) * CH
            d1 = pltpu.async_copy(src_ix.at[pl.ds(e, CH)], idx_v.at[k, 0], sem_i[k])
            d2 = pltpu.async_copy(dst_ix.at[pl.ds(e, CH)], idx_v.at[k, 1], sem_i[k])
            idx_d.append((d1, d2))
        g_d = [None] * G
        s_d = [None] * G

        def start_scatter(k):
            s_d[k] = pltpu.async_copy(
                rows_v.at[k], acc_sh.at[idx_v.at[k, 1]], sem_s[k], add=True)

        for k in range(G):
            idx_d[k][0].wait()
            idx_d[k][1].wait()
            g_d[k] = pltpu.async_copy(g.at[idx_v.at[k, 0]], rows_v.at[k], sem_g[k])
            if k >= 1:
                g_d[k - 1].wait()
                start_scatter(k - 1)
        g_d[G - 1].wait()
        start_scatter(G - 1)
        for k in range(G):
            s_d[k].wait()
        return carry

    lax.fori_loop(0, NT // G, group, 0)
    plsc.subcore_barrier()
    pltpu.sync_copy(acc_sh.at[pl.ds(s * RP, RP)], out.at[c, pl.ds(s * RP, RP)])


def _make_scatter_kernel(d):
    return functools.partial(
        pl.kernel,
        out_type=jax.ShapeDtypeStruct((NC, NP, d), jnp.float32),
        mesh=_MESH,
        scratch_types=[
            pltpu.VMEM((G, 2, CH), jnp.int32),
            pltpu.VMEM((G, CH, d), jnp.float32),
            pltpu.VMEM_SHARED((NP, d), jnp.float32),
        ] + [pltpu.SemaphoreType.DMA] * (3 * G),
    )(_scatter_body)


_scatter_128 = _make_scatter_kernel(D_H1)

RB = 1024


def _mm_scale_body(x_ref, w_ref, dinv_ref, o_ref):
    o_ref[...] = (
        jnp.dot(x_ref[...], w_ref[...], preferred_element_type=jnp.float32)
        * dinv_ref[...]
    )


def _mid_body(p0_ref, p1_ref, g_ref, dinv_ref, b_ref, o_ref):
    h = jnp.maximum(
        (p0_ref[...] + p1_ref[...] + g_ref[...]) * dinv_ref[...] + b_ref[...],
        0.0,
    )
    o_ref[...] = h * dinv_ref[...]


def _final_body(p0_ref, p1_ref, g_ref, dinv_ref, b_ref, w_ref, o_ref):
    acc = p0_ref[...] + p1_ref[...] + g_ref[...]
    o_ref[...] = jnp.maximum(
        jnp.dot(acc, w_ref[...], preferred_element_type=jnp.float32)
        * dinv_ref[...]
        + b_ref[...],
        0.0,
    )


def _rows(bs):
    return pl.BlockSpec((RB, bs), lambda i: (i, 0))


def _full(r, c):
    return pl.BlockSpec((r, c), lambda i: (0, 0))


def _mm_scale(x, w, dinv):
    din, dout = w.shape
    return pl.pallas_call(
        _mm_scale_body,
        grid=(NP // RB,),
        in_specs=[_rows(din), _full(din, dout), _rows(1)],
        out_specs=_rows(dout),
        out_shape=jax.ShapeDtypeStruct((NP, dout), jnp.float32),
    )(x, w, dinv)


def _mid(p0, p1, g, dinv, b):
    d = g.shape[1]
    return pl.pallas_call(
        _mid_body,
        grid=(NP // RB,),
        in_specs=[_rows(d), _rows(d), _rows(d), _rows(1), _full(1, d)],
        out_specs=_rows(d),
        out_shape=jax.ShapeDtypeStruct((NP, d), jnp.float32),
    )(p0, p1, g, dinv, b)


def _final(p0, p1, g, dinv, b, w):
    din, dout = w.shape
    return pl.pallas_call(
        _final_body,
        grid=(NP // RB,),
        in_specs=[_rows(din), _rows(din), _rows(din), _rows(1),
                  _full(1, dout), _full(din, dout)],
        out_specs=_rows(dout),
        out_shape=jax.ShapeDtypeStruct((NP, dout), jnp.float32),
    )(p0, p1, g, dinv, b, w)


def kernel(x, edge_index, W1, b1, W2, b2):
    x_p = jnp.zeros((NP, D_IN), jnp.float32).at[:N].set(x)
    pad = (jnp.arange(EPA - E, dtype=jnp.int32) % (NP - N)) + N
    src_p = jnp.concatenate([edge_index[0], pad])
    dst_p = jnp.concatenate([edge_index[1], pad])

    zvec = jnp.zeros((NP,), jnp.float32)
    z128 = jnp.zeros((NP, D_H1), jnp.float32)

    degp = _deg_kernel(dst_p, zvec).reshape(NC, NP)
    dinv = lax.rsqrt(degp[0] + degp[1] + 1.0).reshape(NP, 1)

    g1 = _mm_scale(x_p, W1, dinv)
    p1 = _scatter_128(src_p, dst_p, z128, g1)
    u = _mid(p1[0], p1[1], g1, dinv, b1.reshape(1, D_H1))
    p2 = _scatter_128(src_p, dst_p, z128, u)
    z = _final(p2[0], p2[1], u, dinv, b2.reshape(1, D_H2), W2)
    return z[:N]


# speedup vs baseline: 27.7700x; 1.0920x over previous
import functools

import jax
import jax.numpy as jnp
from jax import lax
from jax.experimental import pallas as pl
from jax.experimental.pallas import tpu as pltpu
from jax.experimental.pallas import tpu_sc as plsc

N = 10000
E = 640000
D_IN = 128
D_H1 = 128
D_H2 = 64

NC = 2
NS = 16
NW = NC * NS

CH = 112
G = 3
NT = 180
EP = NW * NT * CH
EPA = EP
NP = 10240
RP = NP // NS

_MESH = plsc.VectorSubcoreMesh(core_axis_name="c", subcore_axis_name="s")


def _deg_body(dst, zvec, out, idx_v, ones_v, deg_sh, *sems):
    sem_i = sems[:4]
    sem_s = sems[4:]
    c = lax.axis_index("c")
    s = lax.axis_index("s")
    w = s * NC + c
    for i in range(CH // 16):
        ones_v[pl.ds(i * 16, 16)] = jnp.ones((16,), jnp.float32)
    assert NT % 4 == 0 and NT % G == 0
    pltpu.sync_copy(zvec.at[pl.ds(s * RP, RP)], deg_sh.at[pl.ds(s * RP, RP)])
    plsc.subcore_barrier()
    base = w * NT * CH

    GD = 4

    def group(tt, carry):
        idx_d = []
        for k in range(GD):
            e = base + (tt * GD + k) * CH
            idx_d.append(pltpu.async_copy(
                dst.at[pl.ds(e, CH)], idx_v.at[k], sem_i[k]))
        s_d = []
        for k in range(GD):
            idx_d[k].wait()
            s_d.append(pltpu.async_copy(
                ones_v, deg_sh.at[idx_v.at[k]], sem_s[k], add=True))
        for k in range(GD):
            s_d[k].wait()
        return carry

    lax.fori_loop(0, NT // GD, group, 0)
    plsc.subcore_barrier()
    pltpu.sync_copy(deg_sh.at[pl.ds(s * RP, RP)],
                    out.at[pl.ds(c * NP + s * RP, RP)])


_deg_kernel = functools.partial(
    pl.kernel,
    out_type=jax.ShapeDtypeStruct((NC * NP,), jnp.float32),
    mesh=_MESH,
    scratch_types=[
        pltpu.VMEM((4, CH), jnp.int32),
        pltpu.VMEM((CH,), jnp.float32),
        pltpu.VMEM_SHARED((NP,), jnp.float32),
    ] + [pltpu.SemaphoreType.DMA] * 8,
)(_deg_body)


def _scatter_body(src_ix, dst_ix, zrows, g, out, idx_v, rows_v, acc_sh, *sems):
    sem_i = sems[:G]
    sem_g = sems[G:2 * G]
    sem_s = sems[2 * G:3 * G]
    c = lax.axis_index("c")
    s = lax.axis_index("s")
    w = s * NC + c
    pltpu.sync_copy(zrows.at[pl.ds(s * RP, RP)], acc_sh.at[pl.ds(s * RP, RP)])
    plsc.subcore_barrier()
    base = w * NT * CH

    def group(tt, carry):
        idx_d = []
        for k in range(G):
            e = base + (tt * G + k) * CH
            d1 = pltpu.async_copy(src_ix.at[pl.ds(e, CH)], idx_v.at[k, 0], sem_i[k])
            d2 = pltpu.async_copy(dst_ix.at[pl.ds(e, CH)], idx_v.at[k, 1], sem_i[k])
            idx_d.append((d1, d2))
        g_d = [None] * G
        s_d = [None] * G

        def start_scatter(k):
            s_d[k] = pltpu.async_copy(
                rows_v.at[k], acc_sh.at[idx_v.at[k, 1]], sem_s[k], add=True)

        for k in range(G):
            idx_d[k][0].wait()
            idx_d[k][1].wait()
            g_d[k] = pltpu.async_copy(g.at[idx_v.at[k, 0]], rows_v.at[k], sem_g[k])
            if k >= 1:
                g_d[k - 1].wait()
                start_scatter(k - 1)
        g_d[G - 1].wait()
        start_scatter(G - 1)
        for k in range(G):
            s_d[k].wait()
        return carry

    lax.fori_loop(0, NT // G, group, 0)
    plsc.subcore_barrier()
    pltpu.sync_copy(acc_sh.at[pl.ds(s * RP, RP)], out.at[c, pl.ds(s * RP, RP)])


def _make_scatter_kernel(d):
    return functools.partial(
        pl.kernel,
        out_type=jax.ShapeDtypeStruct((NC, NP, d), jnp.float32),
        mesh=_MESH,
        scratch_types=[
            pltpu.VMEM((G, 2, CH), jnp.int32),
            pltpu.VMEM((G, CH, d), jnp.float32),
            pltpu.VMEM_SHARED((NP, d), jnp.float32),
        ] + [pltpu.SemaphoreType.DMA] * (3 * G),
    )(_scatter_body)


_scatter_128 = _make_scatter_kernel(D_H1)

RB = 1024


def _mm_scale_body(x_ref, w_ref, dinv_ref, o_ref):
    o_ref[...] = (
        jnp.dot(x_ref[...], w_ref[...], preferred_element_type=jnp.float32)
        * dinv_ref[...]
    )


def _mid_body(p0_ref, p1_ref, g_ref, dinv_ref, b_ref, o_ref):
    h = jnp.maximum(
        (p0_ref[...] + p1_ref[...] + g_ref[...]) * dinv_ref[...] + b_ref[...],
        0.0,
    )
    o_ref[...] = h * dinv_ref[...]


def _final_body(p0_ref, p1_ref, g_ref, dinv_ref, b_ref, w_ref, o_ref):
    acc = p0_ref[...] + p1_ref[...] + g_ref[...]
    o_ref[...] = jnp.maximum(
        jnp.dot(acc, w_ref[...], preferred_element_type=jnp.float32)
        * dinv_ref[...]
        + b_ref[...],
        0.0,
    )


def _rows(bs):
    return pl.BlockSpec((RB, bs), lambda i: (i, 0))


def _full(r, c):
    return pl.BlockSpec((r, c), lambda i: (0, 0))


def _mm_scale(x, w, dinv):
    din, dout = w.shape
    return pl.pallas_call(
        _mm_scale_body,
        grid=(NP // RB,),
        in_specs=[_rows(din), _full(din, dout), _rows(1)],
        out_specs=_rows(dout),
        out_shape=jax.ShapeDtypeStruct((NP, dout), jnp.float32),
    )(x, w, dinv)


def _mid(p0, p1, g, dinv, b):
    d = g.shape[1]
    return pl.pallas_call(
        _mid_body,
        grid=(NP // RB,),
        in_specs=[_rows(d), _rows(d), _rows(d), _rows(1), _full(1, d)],
        out_specs=_rows(d),
        out_shape=jax.ShapeDtypeStruct((NP, d), jnp.float32),
    )(p0, p1, g, dinv, b)


def _final(p0, p1, g, dinv, b, w):
    din, dout = w.shape
    return pl.pallas_call(
        _final_body,
        grid=(NP // RB,),
        in_specs=[_rows(din), _rows(din), _rows(din), _rows(1),
                  _full(1, dout), _full(din, dout)],
        out_specs=_rows(dout),
        out_shape=jax.ShapeDtypeStruct((NP, dout), jnp.float32),
    )(p0, p1, g, dinv, b, w)


def kernel(x, edge_index, W1, b1, W2, b2):
    x_p = jnp.zeros((NP, D_IN), jnp.float32).at[:N].set(x)
    pad = (jnp.arange(EPA - E, dtype=jnp.int32) % (NP - N)) + N
    src_p = jnp.concatenate([edge_index[0], pad])
    dst_p = jnp.concatenate([edge_index[1], pad])

    zvec = jnp.zeros((NP,), jnp.float32)
    z128 = jnp.zeros((NP, D_H1), jnp.float32)

    degp = _deg_kernel(dst_p, zvec).reshape(NC, NP)
    dinv = lax.rsqrt(degp[0] + degp[1] + 1.0).reshape(NP, 1)

    g1 = _mm_scale(x_p, W1, dinv)
    p1 = _scatter_128(src_p, dst_p, z128, g1)
    u = _mid(p1[0], p1[1], g1, dinv, b1.reshape(1, D_H1))
    p2 = _scatter_128(src_p, dst_p, z128, u)
    z = _final(p2[0], p2[1], u, dinv, b2.reshape(1, D_H2), W2)
    return z[:N]


# speedup vs baseline: 36.8319x; 1.3263x over previous
import functools

import jax
import jax.numpy as jnp
from jax import lax
from jax.experimental import pallas as pl
from jax.experimental.pallas import tpu as pltpu
from jax.experimental.pallas import tpu_sc as plsc

N = 10000
E = 640000
D_IN = 128
D_H1 = 128
D_H2 = 64

NC = 2
NS = 16
NW = NC * NS

CH = 112
RING = 3
M = 18
NT = 180
EP = NW * NT * CH
EPA = EP
NP = 10240
RP = NP // NS

_MESH = plsc.VectorSubcoreMesh(core_axis_name="c", subcore_axis_name="s")


_GD = 12


def _deg_body(dst, zvec, out, idx_v, ones_v, deg_sh, *sems):
    sem_i = sems[:_GD]
    sem_s = sems[_GD:]
    c = lax.axis_index("c")
    s = lax.axis_index("s")
    w = s * NC + c
    for i in range(CH // 16):
        ones_v[pl.ds(i * 16, 16)] = jnp.ones((16,), jnp.float32)
    assert NT % _GD == 0 and NT % M == 0
    pltpu.sync_copy(zvec.at[pl.ds(s * RP, RP)], deg_sh.at[pl.ds(s * RP, RP)])
    plsc.subcore_barrier()
    base = w * NT * CH

    def group(tt, carry):
        idx_d = []
        for k in range(_GD):
            e = base + (tt * _GD + k) * CH
            idx_d.append(pltpu.async_copy(
                dst.at[pl.ds(e, CH)], idx_v.at[k], sem_i[k]))
        s_d = []
        for k in range(_GD):
            idx_d[k].wait()
            s_d.append(pltpu.async_copy(
                ones_v, deg_sh.at[idx_v.at[k]], sem_s[k], add=True))
        for k in range(_GD):
            s_d[k].wait()
        return carry

    lax.fori_loop(0, NT // _GD, group, 0)
    plsc.subcore_barrier()
    pltpu.sync_copy(deg_sh.at[pl.ds(s * RP, RP)],
                    out.at[pl.ds(c * NP + s * RP, RP)])


_deg_kernel = functools.partial(
    pl.kernel,
    out_type=jax.ShapeDtypeStruct((NC * NP,), jnp.float32),
    mesh=_MESH,
    scratch_types=[
        pltpu.VMEM((_GD, CH), jnp.int32),
        pltpu.VMEM((CH,), jnp.float32),
        pltpu.VMEM_SHARED((NP,), jnp.float32),
    ] + [pltpu.SemaphoreType.DMA] * (2 * _GD),
)(_deg_body)


def _scatter_body(src_ix, dst_ix, zrows, g, out, idx_v, rows_v, acc_sh, *sems):
    sem_i = sems[:M]
    sem_g = sems[M:M + RING]
    sem_s = sems[M + RING:M + 2 * RING]
    c = lax.axis_index("c")
    s = lax.axis_index("s")
    w = s * NC + c
    pltpu.sync_copy(zrows.at[pl.ds(s * RP, RP)], acc_sh.at[pl.ds(s * RP, RP)])
    plsc.subcore_barrier()
    base = w * NT * CH

    def group(tt, carry):
        idx_d = []
        for k in range(M):
            e = base + (tt * M + k) * CH
            d1 = pltpu.async_copy(src_ix.at[pl.ds(e, CH)], idx_v.at[k, 0], sem_i[k])
            d2 = pltpu.async_copy(dst_ix.at[pl.ds(e, CH)], idx_v.at[k, 1], sem_i[k])
            idx_d.append((d1, d2))
        g_d = [None] * M
        s_d = [None] * M

        def start_scatter(k):
            s_d[k] = pltpu.async_copy(
                rows_v.at[k % RING], acc_sh.at[idx_v.at[k, 1]],
                sem_s[k % RING], add=True)

        for k in range(M):
            idx_d[k][0].wait()
            idx_d[k][1].wait()
            if k >= RING:
                s_d[k - RING].wait()
            g_d[k] = pltpu.async_copy(
                g.at[idx_v.at[k, 0]], rows_v.at[k % RING], sem_g[k % RING])
            if k >= 1:
                g_d[k - 1].wait()
                start_scatter(k - 1)
        g_d[M - 1].wait()
        start_scatter(M - 1)
        for k in range(M - RING, M):
            s_d[k].wait()
        return carry

    lax.fori_loop(0, NT // M, group, 0)
    plsc.subcore_barrier()
    pltpu.sync_copy(acc_sh.at[pl.ds(s * RP, RP)], out.at[c, pl.ds(s * RP, RP)])


def _make_scatter_kernel(d):
    return functools.partial(
        pl.kernel,
        out_type=jax.ShapeDtypeStruct((NC, NP, d), jnp.float32),
        mesh=_MESH,
        scratch_types=[
            pltpu.VMEM((M, 2, CH), jnp.int32),
            pltpu.VMEM((RING, CH, d), jnp.float32),
            pltpu.VMEM_SHARED((NP, d), jnp.float32),
        ] + [pltpu.SemaphoreType.DMA] * (M + 2 * RING),
    )(_scatter_body)


_scatter_128 = _make_scatter_kernel(D_H1)

RB = 1024


def _mm_scale_body(x_ref, w_ref, dinv_ref, o_ref):
    o_ref[...] = (
        jnp.dot(x_ref[...], w_ref[...], preferred_element_type=jnp.float32)
        * dinv_ref[...]
    )


def _mid_body(p0_ref, p1_ref, g_ref, dinv_ref, b_ref, o_ref):
    h = jnp.maximum(
        (p0_ref[...] + p1_ref[...] + g_ref[...]) * dinv_ref[...] + b_ref[...],
        0.0,
    )
    o_ref[...] = h * dinv_ref[...]


def _final_body(p0_ref, p1_ref, g_ref, dinv_ref, b_ref, w_ref, o_ref):
    acc = p0_ref[...] + p1_ref[...] + g_ref[...]
    o_ref[...] = jnp.maximum(
        jnp.dot(acc, w_ref[...], preferred_element_type=jnp.float32)
        * dinv_ref[...]
        + b_ref[...],
        0.0,
    )


def _rows(bs):
    return pl.BlockSpec((RB, bs), lambda i: (i, 0))


def _full(r, c):
    return pl.BlockSpec((r, c), lambda i: (0, 0))


def _mm_scale(x, w, dinv):
    din, dout = w.shape
    return pl.pallas_call(
        _mm_scale_body,
        grid=(NP // RB,),
        in_specs=[_rows(din), _full(din, dout), _rows(1)],
        out_specs=_rows(dout),
        out_shape=jax.ShapeDtypeStruct((NP, dout), jnp.float32),
    )(x, w, dinv)


def _mid(p0, p1, g, dinv, b):
    d = g.shape[1]
    return pl.pallas_call(
        _mid_body,
        grid=(NP // RB,),
        in_specs=[_rows(d), _rows(d), _rows(d), _rows(1), _full(1, d)],
        out_specs=_rows(d),
        out_shape=jax.ShapeDtypeStruct((NP, d), jnp.float32),
    )(p0, p1, g, dinv, b)


def _final(p0, p1, g, dinv, b, w):
    din, dout = w.shape
    return pl.pallas_call(
        _final_body,
        grid=(NP // RB,),
        in_specs=[_rows(din), _rows(din), _rows(din), _rows(1),
                  _full(1, dout), _full(din, dout)],
        out_specs=_rows(dout),
        out_shape=jax.ShapeDtypeStruct((NP, dout), jnp.float32),
    )(p0, p1, g, dinv, b, w)


def kernel(x, edge_index, W1, b1, W2, b2):
    x_p = jnp.zeros((NP, D_IN), jnp.float32).at[:N].set(x)
    pad = (jnp.arange(EPA - E, dtype=jnp.int32) % (NP - N)) + N
    src_p = jnp.concatenate([edge_index[0], pad])
    dst_p = jnp.concatenate([edge_index[1], pad])

    zvec = jnp.zeros((NP,), jnp.float32)
    z128 = jnp.zeros((NP, D_H1), jnp.float32)

    degp = _deg_kernel(dst_p, zvec).reshape(NC, NP)
    dinv = lax.rsqrt(degp[0] + degp[1] + 1.0).reshape(NP, 1)

    g1 = _mm_scale(x_p, W1, dinv)
    p1 = _scatter_128(src_p, dst_p, z128, g1)
    u = _mid(p1[0], p1[1], g1, dinv, b1.reshape(1, D_H1))
    p2 = _scatter_128(src_p, dst_p, z128, u)
    z = _final(p2[0], p2[1], u, dinv, b2.reshape(1, D_H2), W2)
    return z[:N]


# speedup vs baseline: 41.2327x; 1.1195x over previous
import functools

import jax
import jax.numpy as jnp
from jax import lax
from jax.experimental import pallas as pl
from jax.experimental.pallas import tpu as pltpu
from jax.experimental.pallas import tpu_sc as plsc

N = 10000
E = 640000
D_IN = 128
D_H1 = 128
D_H2 = 64

NC = 2
NS = 16
NW = NC * NS

CH = 112
RING = 3
M = 18
NT = 180
EP = NW * NT * CH
EPA = EP
NP = 10240
RP = NP // NS

_MESH = plsc.VectorSubcoreMesh(core_axis_name="c", subcore_axis_name="s")


_GD = 12


def _deg_body(dst, zvec, out, idx_v, ones_v, deg_sh, *sems):
    sem_i = sems[:_GD]
    sem_s = sems[_GD:]
    c = lax.axis_index("c")
    s = lax.axis_index("s")
    w = s * NC + c
    for i in range(CH // 16):
        ones_v[pl.ds(i * 16, 16)] = jnp.ones((16,), jnp.float32)
    assert NT % _GD == 0 and NT % M == 0
    pltpu.sync_copy(zvec.at[pl.ds(s * RP, RP)], deg_sh.at[pl.ds(s * RP, RP)])
    plsc.subcore_barrier()
    base = w * NT * CH

    def group(tt, carry):
        idx_d = []
        for k in range(_GD):
            e = base + (tt * _GD + k) * CH
            idx_d.append(pltpu.async_copy(
                dst.at[pl.ds(e, CH)], idx_v.at[k], sem_i[k]))
        s_d = []
        for k in range(_GD):
            idx_d[k].wait()
            s_d.append(pltpu.async_copy(
                ones_v, deg_sh.at[idx_v.at[k]], sem_s[k], add=True))
        for k in range(_GD):
            s_d[k].wait()
        return carry

    lax.fori_loop(0, NT // _GD, group, 0)
    plsc.subcore_barrier()
    pltpu.sync_copy(deg_sh.at[pl.ds(s * RP, RP)],
                    out.at[pl.ds(c * NP + s * RP, RP)])


_deg_kernel = functools.partial(
    pl.kernel,
    out_type=jax.ShapeDtypeStruct((NC * NP,), jnp.float32),
    mesh=_MESH,
    scratch_types=[
        pltpu.VMEM((_GD, CH), jnp.int32),
        pltpu.VMEM((CH,), jnp.float32),
        pltpu.VMEM_SHARED((NP,), jnp.float32),
    ] + [pltpu.SemaphoreType.DMA] * (2 * _GD),
)(_deg_body)


def _scatter_body(ring, src_ix, dst_ix, zrows, g, out, idx_v, rows_v, acc_sh, *sems):
    sem_i = sems[:M]
    sem_g = sems[M:M + ring]
    sem_s = sems[M + ring:M + 2 * ring]
    c = lax.axis_index("c")
    s = lax.axis_index("s")
    w = s * NC + c
    pltpu.sync_copy(zrows.at[pl.ds(s * RP, RP)], acc_sh.at[pl.ds(s * RP, RP)])
    plsc.subcore_barrier()
    base = w * NT * CH

    def group(tt, carry):
        idx_d = []
        for k in range(M):
            e = base + (tt * M + k) * CH
            d1 = pltpu.async_copy(src_ix.at[pl.ds(e, CH)], idx_v.at[k, 0], sem_i[k])
            d2 = pltpu.async_copy(dst_ix.at[pl.ds(e, CH)], idx_v.at[k, 1], sem_i[k])
            idx_d.append((d1, d2))
        g_d = [None] * M
        s_d = [None] * M

        def start_scatter(k):
            s_d[k] = pltpu.async_copy(
                rows_v.at[k % ring], acc_sh.at[idx_v.at[k, 1]],
                sem_s[k % ring], add=True)

        for k in range(M):
            idx_d[k][0].wait()
            idx_d[k][1].wait()
            if k >= ring:
                s_d[k - ring].wait()
            g_d[k] = pltpu.async_copy(
                g.at[idx_v.at[k, 0]], rows_v.at[k % ring], sem_g[k % ring])
            if k >= 1:
                g_d[k - 1].wait()
                start_scatter(k - 1)
        g_d[M - 1].wait()
        start_scatter(M - 1)
        for k in range(M - ring, M):
            s_d[k].wait()
        return carry

    lax.fori_loop(0, NT // M, group, 0)
    plsc.subcore_barrier()
    pltpu.sync_copy(acc_sh.at[pl.ds(s * RP, RP)], out.at[c, pl.ds(s * RP, RP)])


def _make_scatter_kernel(d, ring, tc_tiling):
    return functools.partial(
        pl.kernel,
        out_type=jax.ShapeDtypeStruct((NC, NP, d), jnp.float32),
        mesh=_MESH,
        scratch_types=[
            pltpu.VMEM((M, 2, CH), jnp.int32),
            pltpu.VMEM((ring, CH, d), jnp.float32),
            pltpu.VMEM_SHARED((NP, d), jnp.float32),
        ] + [pltpu.SemaphoreType.DMA] * (M + 2 * ring),
        compiler_params=pltpu.CompilerParams(use_tc_tiling_on_sc=tc_tiling),
    )(functools.partial(_scatter_body, ring))


_scatter_128 = _make_scatter_kernel(D_H1, RING, True)
_scatter_64 = _make_scatter_kernel(D_H2, 4, False)

RB = 1024


def _mm_scale_body(x_ref, w_ref, dinv_ref, o_ref):
    o_ref[...] = (
        jnp.dot(x_ref[...], w_ref[...], preferred_element_type=jnp.float32)
        * dinv_ref[...]
    )


def _mid_body(p0_ref, p1_ref, g_ref, dinv_ref, b_ref, w_ref, o_ref):
    h = jnp.maximum(
        (p0_ref[...] + p1_ref[...] + g_ref[...]) * dinv_ref[...] + b_ref[...],
        0.0,
    )
    o_ref[...] = (
        jnp.dot(h, w_ref[...], preferred_element_type=jnp.float32)
        * dinv_ref[...]
    )


def _final_body(p0_ref, p1_ref, g_ref, dinv_ref, b_ref, o_ref):
    o_ref[...] = jnp.maximum(
        (p0_ref[...] + p1_ref[...] + g_ref[...]) * dinv_ref[...] + b_ref[...],
        0.0,
    )


def _rows(bs):
    return pl.BlockSpec((RB, bs), lambda i: (i, 0))


def _full(r, c):
    return pl.BlockSpec((r, c), lambda i: (0, 0))


def _mm_scale(x, w, dinv):
    din, dout = w.shape
    return pl.pallas_call(
        _mm_scale_body,
        grid=(NP // RB,),
        in_specs=[_rows(din), _full(din, dout), _rows(1)],
        out_specs=_rows(dout),
        out_shape=jax.ShapeDtypeStruct((NP, dout), jnp.float32),
    )(x, w, dinv)


def _mid(p0, p1, g, dinv, b, w):
    din, dout = w.shape
    return pl.pallas_call(
        _mid_body,
        grid=(NP // RB,),
        in_specs=[_rows(din), _rows(din), _rows(din), _rows(1),
                  _full(1, din), _full(din, dout)],
        out_specs=_rows(dout),
        out_shape=jax.ShapeDtypeStruct((NP, dout), jnp.float32),
    )(p0, p1, g, dinv, b, w)


def _final(p0, p1, g, dinv, b):
    d = g.shape[1]
    return pl.pallas_call(
        _final_body,
        grid=(NP // RB,),
        in_specs=[_rows(d), _rows(d), _rows(d), _rows(1), _full(1, d)],
        out_specs=_rows(d),
        out_shape=jax.ShapeDtypeStruct((NP, d), jnp.float32),
    )(p0, p1, g, dinv, b)


def kernel(x, edge_index, W1, b1, W2, b2):
    x_p = jnp.zeros((NP, D_IN), jnp.float32).at[:N].set(x)
    pad = (jnp.arange(EPA - E, dtype=jnp.int32) % (NP - N)) + N
    src_p = jnp.concatenate([edge_index[0], pad])
    dst_p = jnp.concatenate([edge_index[1], pad])

    zvec = jnp.zeros((NP,), jnp.float32)
    z128 = jnp.zeros((NP, D_H1), jnp.float32)
    z64 = jnp.zeros((NP, D_H2), jnp.float32)

    degp = _deg_kernel(dst_p, zvec).reshape(NC, NP)
    dinv = lax.rsqrt(degp[0] + degp[1] + 1.0).reshape(NP, 1)

    g1 = _mm_scale(x_p, W1, dinv)
    p1 = _scatter_128(src_p, dst_p, z128, g1)
    g2 = _mid(p1[0], p1[1], g1, dinv, b1.reshape(1, D_H1), W2)
    p2 = _scatter_64(src_p, dst_p, z64, g2)
    z = _final(p2[0], p2[1], g2, dinv, b2.reshape(1, D_H2))
    return z[:N]


# speedup vs baseline: 43.4285x; 1.0533x over previous
import functools

import jax
import jax.numpy as jnp
from jax import lax
from jax.experimental import pallas as pl
from jax.experimental.pallas import tpu as pltpu
from jax.experimental.pallas import tpu_sc as plsc

N = 10000
E = 640000
D_IN = 128
D_H1 = 128
D_H2 = 64

NC = 2
NS = 16
NW = NC * NS

CH = 112
RING = 3
M = 18
NT = 180
EP = NW * NT * CH
EPA = EP
NP = 10240
RP = NP // NS

_MESH = plsc.VectorSubcoreMesh(core_axis_name="c", subcore_axis_name="s")


_GD = 12


def _deg_body(dst, zvec, out, idx_v, ones_v, deg_sh, *sems):
    sem_i = sems[:_GD]
    sem_s = sems[_GD:]
    c = lax.axis_index("c")
    s = lax.axis_index("s")
    w = s * NC + c
    for i in range(CH // 16):
        ones_v[pl.ds(i * 16, 16)] = jnp.ones((16,), jnp.float32)
    assert NT % _GD == 0 and NT % M == 0
    pltpu.sync_copy(zvec.at[pl.ds(s * RP, RP)], deg_sh.at[pl.ds(s * RP, RP)])
    plsc.subcore_barrier()
    base = w * NT * CH

    def group(tt, carry):
        idx_d = []
        for k in range(_GD):
            e = base + (tt * _GD + k) * CH
            idx_d.append(pltpu.async_copy(
                dst.at[pl.ds(e, CH)], idx_v.at[k], sem_i[k]))
        s_d = []
        for k in range(_GD):
            idx_d[k].wait()
            s_d.append(pltpu.async_copy(
                ones_v, deg_sh.at[idx_v.at[k]], sem_s[k], add=True))
        for k in range(_GD):
            s_d[k].wait()
        return carry

    lax.fori_loop(0, NT // _GD, group, 0)
    plsc.subcore_barrier()
    pltpu.sync_copy(deg_sh.at[pl.ds(s * RP, RP)],
                    out.at[pl.ds(c * NP + s * RP, RP)])


_deg_kernel = functools.partial(
    pl.kernel,
    out_type=jax.ShapeDtypeStruct((NC * NP,), jnp.float32),
    mesh=_MESH,
    scratch_types=[
        pltpu.VMEM((_GD, CH), jnp.int32),
        pltpu.VMEM((CH,), jnp.float32),
        pltpu.VMEM_SHARED((NP,), jnp.float32),
    ] + [pltpu.SemaphoreType.DMA] * (2 * _GD),
)(_deg_body)


def _scatter_body(ring, src_ix, dst_ix, zrows, g, out, idx_v, rows_v, acc_sh, *sems):
    sem_i = sems[:M]
    sem_g = sems[M:M + ring]
    sem_s = sems[M + ring:M + 2 * ring]
    c = lax.axis_index("c")
    s = lax.axis_index("s")
    w = s * NC + c
    pltpu.sync_copy(zrows.at[pl.ds(s * RP, RP)], acc_sh.at[pl.ds(s * RP, RP)])
    plsc.subcore_barrier()
    base = w * NT * CH

    def group(tt, carry):
        idx_d = []
        for k in range(M):
            e = base + (tt * M + k) * CH
            d1 = pltpu.async_copy(src_ix.at[pl.ds(e, CH)], idx_v.at[k, 0], sem_i[k])
            d2 = pltpu.async_copy(dst_ix.at[pl.ds(e, CH)], idx_v.at[k, 1], sem_i[k])
            idx_d.append((d1, d2))
        g_d = [None] * M
        s_d = [None] * M

        def start_scatter(k):
            s_d[k] = pltpu.async_copy(
                rows_v.at[k % ring], acc_sh.at[idx_v.at[k, 1]],
                sem_s[k % ring], add=True)

        lag = ring - 1
        for k in range(M):
            idx_d[k][0].wait()
            idx_d[k][1].wait()
            if k >= ring:
                s_d[k - ring].wait()
            g_d[k] = pltpu.async_copy(
                g.at[idx_v.at[k, 0]], rows_v.at[k % ring], sem_g[k % ring])
            if k >= lag:
                g_d[k - lag].wait()
                start_scatter(k - lag)
        for k in range(M - lag, M):
            g_d[k].wait()
            start_scatter(k)
        for k in range(M - ring, M):
            s_d[k].wait()
        return carry

    lax.fori_loop(0, NT // M, group, 0)
    plsc.subcore_barrier()
    pltpu.sync_copy(acc_sh.at[pl.ds(s * RP, RP)], out.at[c, pl.ds(s * RP, RP)])


def _make_scatter_kernel(d, ring, tc_tiling):
    return functools.partial(
        pl.kernel,
        out_type=jax.ShapeDtypeStruct((NC, NP, d), jnp.float32),
        mesh=_MESH,
        scratch_types=[
            pltpu.VMEM((M, 2, CH), jnp.int32),
            pltpu.VMEM((ring, CH, d), jnp.float32),
            pltpu.VMEM_SHARED((NP, d), jnp.float32),
        ] + [pltpu.SemaphoreType.DMA] * (M + 2 * ring),
        compiler_params=pltpu.CompilerParams(use_tc_tiling_on_sc=tc_tiling),
    )(functools.partial(_scatter_body, ring))


_scatter_128 = _make_scatter_kernel(D_H1, RING, True)
_scatter_64 = _make_scatter_kernel(D_H2, 4, False)

RB = 1024


def _mm_scale_body(x_ref, w_ref, dinv_ref, o_ref):
    o_ref[...] = (
        jnp.dot(x_ref[...], w_ref[...], preferred_element_type=jnp.float32)
        * dinv_ref[...]
    )


def _mid_body(p0_ref, p1_ref, g_ref, dinv_ref, b_ref, w_ref, o_ref):
    h = jnp.maximum(
        (p0_ref[...] + p1_ref[...] + g_ref[...]) * dinv_ref[...] + b_ref[...],
        0.0,
    )
    o_ref[...] = (
        jnp.dot(h, w_ref[...], preferred_element_type=jnp.float32)
        * dinv_ref[...]
    )


def _final_body(p0_ref, p1_ref, g_ref, dinv_ref, b_ref, o_ref):
    o_ref[...] = jnp.maximum(
        (p0_ref[...] + p1_ref[...] + g_ref[...]) * dinv_ref[...] + b_ref[...],
        0.0,
    )


def _rows(bs):
    return pl.BlockSpec((RB, bs), lambda i: (i, 0))


def _full(r, c):
    return pl.BlockSpec((r, c), lambda i: (0, 0))


def _mm_scale(x, w, dinv):
    din, dout = w.shape
    return pl.pallas_call(
        _mm_scale_body,
        grid=(NP // RB,),
        in_specs=[_rows(din), _full(din, dout), _rows(1)],
        out_specs=_rows(dout),
        out_shape=jax.ShapeDtypeStruct((NP, dout), jnp.float32),
    )(x, w, dinv)


def _mid(p0, p1, g, dinv, b, w):
    din, dout = w.shape
    return pl.pallas_call(
        _mid_body,
        grid=(NP // RB,),
        in_specs=[_rows(din), _rows(din), _rows(din), _rows(1),
                  _full(1, din), _full(din, dout)],
        out_specs=_rows(dout),
        out_shape=jax.ShapeDtypeStruct((NP, dout), jnp.float32),
    )(p0, p1, g, dinv, b, w)


def _final(p0, p1, g, dinv, b):
    d = g.shape[1]
    return pl.pallas_call(
        _final_body,
        grid=(NP // RB,),
        in_specs=[_rows(d), _rows(d), _rows(d), _rows(1), _full(1, d)],
        out_specs=_rows(d),
        out_shape=jax.ShapeDtypeStruct((NP, d), jnp.float32),
    )(p0, p1, g, dinv, b)


def kernel(x, edge_index, W1, b1, W2, b2):
    x_p = jnp.zeros((NP, D_IN), jnp.float32).at[:N].set(x)
    pad = (jnp.arange(EPA - E, dtype=jnp.int32) % (NP - N)) + N
    src_p = jnp.concatenate([edge_index[0], pad])
    dst_p = jnp.concatenate([edge_index[1], pad])

    zvec = jnp.zeros((NP,), jnp.float32)
    z128 = jnp.zeros((NP, D_H1), jnp.float32)
    z64 = jnp.zeros((NP, D_H2), jnp.float32)

    degp = _deg_kernel(dst_p, zvec).reshape(NC, NP)
    dinv = lax.rsqrt(degp[0] + degp[1] + 1.0).reshape(NP, 1)

    g1 = _mm_scale(x_p, W1, dinv)
    p1 = _scatter_128(src_p, dst_p, z128, g1)
    g2 = _mid(p1[0], p1[1], g1, dinv, b1.reshape(1, D_H1), W2)
    p2 = _scatter_64(src_p, dst_p, z64, g2)
    z = _final(p2[0], p2[1], g2, dinv, b2.reshape(1, D_H2))
    return z[:N]
